# baseline (speedup 1.0000x reference)
import jax
import jax.numpy as jnp
from jax import lax
from jax.experimental import pallas as pl
from jax.experimental.pallas import tpu as pltpu
from jax.experimental.pallas import tpu_sc as plsc

N_SPINS = 20
B_ROWS = 16384
NUM_CORES = 2
NUM_SUBCORES = 16
LANES = 16
NUM_WORKERS = NUM_CORES * NUM_SUBCORES
BPW = B_ROWS // NUM_WORKERS
GCHUNK = 128
NGC = BPW // GCHUNK
IDX_COLS = 128
IDX_ROWS = B_ROWS // IDX_COLS
ROWS_PW = IDX_ROWS // NUM_WORKERS

_LN2 = 0.6931471805599453
_SQRT2 = 1.4142135623730951


def _pack_body(x_ref, o_ref):
    xb = x_ref[...].astype(jnp.bfloat16)
    shifts = (N_SPINS - 1) - lax.broadcasted_iota(jnp.int32, (N_SPINS, 1), 0)
    w = (1 << shifts).astype(jnp.bfloat16)
    acc = lax.dot_general(xb, w, (((1,), (0,)), ((), ())),
                          preferred_element_type=jnp.float32)
    o_ref[...] = acc.astype(jnp.int32)


def _ln16(v):
    bits = plsc.bitcast(v, jnp.int32)
    e = ((bits >> 23) & 0xFF) - 127
    m = plsc.bitcast((bits & 0x007FFFFF) | 0x3F800000, jnp.float32)
    big = m > _SQRT2
    m = jnp.where(big, m * 0.5, m)
    e = e + big.astype(jnp.int32)
    s = (m - 1.0) / (m + 1.0)
    z = s * s
    p = 2.0 * s * (1.0 + z * (0.3333333333 + z * (0.2 + z * 0.1428571429)))
    return p + e.astype(jnp.float32) * _LN2


def _body(idx_hbm, aux_hbm, out_hbm, idx_v, vals_v, out_v,
          gs0, gs1, gs2, gs3, wsem):
    wid = lax.axis_index("s") * NUM_CORES + lax.axis_index("c")
    base = wid * BPW

    pltpu.sync_copy(idx_hbm.at[pl.ds(wid * ROWS_PW, ROWS_PW), :], idx_v)

    gsems = [gs0, gs1, gs2, gs3]
    gcopies = [
        pltpu.async_copy(aux_hbm.at[idx_v.at[j]],
                         vals_v.at[pl.ds(j * GCHUNK, GCHUNK)], gsems[j])
        for j in range(NGC)
    ]

    wcopies = []
    for j in range(NGC):
        gcopies[j].wait()

        def log_chunk(c, _):
            off = j * GCHUNK + c * LANES
            out_v[pl.ds(off, LANES)] = _ln16(vals_v[pl.ds(off, LANES)])
            return 0

        lax.fori_loop(0, GCHUNK // LANES, log_chunk, 0)
        wcopies.append(
            pltpu.async_copy(out_v.at[pl.ds(j * GCHUNK, GCHUNK)],
                             out_hbm.at[pl.ds(base + j * GCHUNK, GCHUNK)],
                             wsem))
    for cp in wcopies:
        cp.wait()


def kernel(x, aux, j1):
    del j1
    idx = pl.pallas_call(
        _pack_body,
        out_shape=jax.ShapeDtypeStruct((B_ROWS, 1), jnp.int32),
    )(x)
    idx2 = idx.reshape(IDX_ROWS, IDX_COLS)

    mesh = plsc.VectorSubcoreMesh(
        core_axis_name="c", subcore_axis_name="s",
        num_cores=NUM_CORES, num_subcores=NUM_SUBCORES)
    run = pl.kernel(
        _body,
        out_type=jax.ShapeDtypeStruct((B_ROWS,), jnp.float32),
        mesh=mesh,
        compiler_params=pltpu.CompilerParams(
            needs_layout_passes=False,
            disable_bounds_checks=True,
            disable_semaphore_checks=True,
            skip_device_barrier=True,
            use_tc_tiling_on_sc=True,
        ),
        scratch_types=[
            pltpu.VMEM((ROWS_PW, IDX_COLS), jnp.int32),
            pltpu.VMEM((BPW,), jnp.float32),
            pltpu.VMEM((BPW,), jnp.float32),
            pltpu.SemaphoreType.DMA,
            pltpu.SemaphoreType.DMA,
            pltpu.SemaphoreType.DMA,
            pltpu.SemaphoreType.DMA,
            pltpu.SemaphoreType.DMA,
        ],
    )
    return run(idx2, aux)


# speedup vs baseline: 1.1892x; 1.1892x over previous
import jax
import jax.numpy as jnp
from jax import lax
from jax.experimental import pallas as pl
from jax.experimental.pallas import tpu as pltpu
from jax.experimental.pallas import tpu_sc as plsc

N_SPINS = 20
B_ROWS = 16384
NUM_CORES = 2
NUM_SUBCORES = 16
LANES = 16
NUM_WORKERS = NUM_CORES * NUM_SUBCORES
BPW = B_ROWS // NUM_WORKERS
GCHUNK = 128
NGC = BPW // GCHUNK

_LN2 = 0.6931471805599453
_SQRT2 = 1.4142135623730951


def _ln16(v):
    bits = plsc.bitcast(v, jnp.int32)
    e = ((bits >> 23) & 0xFF) - 127
    m = plsc.bitcast((bits & 0x007FFFFF) | 0x3F800000, jnp.float32)
    big = m > _SQRT2
    m = jnp.where(big, m * 0.5, m)
    e = e + big.astype(jnp.int32)
    s = (m - 1.0) / (m + 1.0)
    z = s * s
    p = 2.0 * s * (1.0 + z * (0.3333333333 + z * (0.2 + z * 0.1428571429)))
    return p + e.astype(jnp.float32) * _LN2


def _body(x_hbm, aux_hbm, out_hbm, x_v, idx_v, vals_v, out_v,
          gs0, gs1, gs2, gs3, wsem):
    wid = lax.axis_index("s") * NUM_CORES + lax.axis_index("c")
    base = wid * BPW

    pltpu.sync_copy(x_hbm.at[pl.ds(base, BPW), :], x_v)

    col_ids = [jnp.full((LANES,), i, jnp.int32) for i in range(N_SPINS)]
    lane_iota = lax.iota(jnp.int32, LANES)
    gsems = [gs0, gs1, gs2, gs3]

    gcopies = []
    for j in range(NGC):
        def chunk(c2, _):
            rows = lane_iota + (j * GCHUNK + c2 * LANES)
            acc = jnp.zeros((LANES,), jnp.int32)
            for i in range(N_SPINS):
                bit = plsc.load_gather(x_v, [rows, col_ids[i]])
                acc = acc + acc + bit
            idx_v[j, pl.ds(c2 * LANES, LANES)] = acc
            return 0

        lax.fori_loop(0, GCHUNK // LANES, chunk, 0)
        gcopies.append(
            pltpu.async_copy(aux_hbm.at[idx_v.at[j]],
                             vals_v.at[pl.ds(j * GCHUNK, GCHUNK)], gsems[j]))

    wcopies = []
    for j in range(NGC):
        gcopies[j].wait()

        def log_chunk(c, _):
            off = j * GCHUNK + c * LANES
            out_v[pl.ds(off, LANES)] = _ln16(vals_v[pl.ds(off, LANES)])
            return 0

        lax.fori_loop(0, GCHUNK // LANES, log_chunk, 0)
        wcopies.append(
            pltpu.async_copy(out_v.at[pl.ds(j * GCHUNK, GCHUNK)],
                             out_hbm.at[pl.ds(base + j * GCHUNK, GCHUNK)],
                             wsem))
    for cp in wcopies:
        cp.wait()


def kernel(x, aux, j1):
    del j1
    mesh = plsc.VectorSubcoreMesh(
        core_axis_name="c", subcore_axis_name="s",
        num_cores=NUM_CORES, num_subcores=NUM_SUBCORES)
    run = pl.kernel(
        _body,
        out_type=jax.ShapeDtypeStruct((B_ROWS,), jnp.float32),
        mesh=mesh,
        compiler_params=pltpu.CompilerParams(
            needs_layout_passes=False,
            disable_bounds_checks=True,
            disable_semaphore_checks=True,
            skip_device_barrier=True,
            use_tc_tiling_on_sc=True,
        ),
        scratch_types=[
            pltpu.VMEM((BPW, N_SPINS), jnp.int32),
            pltpu.VMEM((NGC, GCHUNK), jnp.int32),
            pltpu.VMEM((BPW,), jnp.float32),
            pltpu.VMEM((BPW,), jnp.float32),
            pltpu.SemaphoreType.DMA,
            pltpu.SemaphoreType.DMA,
            pltpu.SemaphoreType.DMA,
            pltpu.SemaphoreType.DMA,
            pltpu.SemaphoreType.DMA,
        ],
    )
    return run(x, aux)
